# baseline (speedup 1.0000x reference)
import functools

import jax
import jax.numpy as jnp
from jax import lax
from jax.experimental import pallas as pl
from jax.experimental.pallas import tpu as pltpu
from jax.experimental.pallas import tpu_sc as plsc

EMBEDDING_DIM = 32
M_TILE = 1024

_SC_CORES = 2
_SC_SUBCORES = 16
_NW = _SC_CORES * _SC_SUBCORES
_IDX_CHUNK = 128


_ROW_CHUNK = 64
_COL_GROUP = 128


def _argmin_body(xn_ref, x_ref, e_ref, idx_ref, table_ref, en_scr):
    @pl.when(pl.program_id(0) == 0)
    def _():
        e = e_ref[...]
        en_scr[...] = jnp.sum(e * e, axis=0, keepdims=True)
        table_ref[...] = e.T

    sim2 = jnp.dot(
        x_ref[...] * -2.0, e_ref[...], preferred_element_type=jnp.float32
    )
    n = sim2.shape[1]
    groups = n // _COL_GROUP
    en = en_scr[...]
    lanef = lax.broadcasted_iota(jnp.int32, (1, _COL_GROUP), 1).astype(
        jnp.float32
    )
    chunks = []
    for c in range(M_TILE // _ROW_CHUNK):
        r0 = c * _ROW_CHUNK
        xn_c = xn_ref[pl.ds(r0, _ROW_CHUNK), :]
        runmin = (xn_c + en[:, :_COL_GROUP]) + sim2[r0:r0 + _ROW_CHUNK, :_COL_GROUP]
        runidx = jnp.zeros((_ROW_CHUNK, _COL_GROUP), jnp.float32)
        for j in range(1, groups):
            c0 = j * _COL_GROUP
            d = (xn_c + en[:, c0:c0 + _COL_GROUP]) + sim2[
                r0:r0 + _ROW_CHUNK, c0:c0 + _COL_GROUP
            ]
            better = d < runmin
            runmin = jnp.where(better, d, runmin)
            runidx = jnp.where(better, jnp.float32(j), runidx)
        m = jnp.min(runmin, axis=1, keepdims=True)
        cand = jnp.where(
            runmin == m, runidx * float(_COL_GROUP) + lanef, jnp.float32(3.0e38)
        )
        idxf = jnp.min(cand, axis=1, keepdims=True)
        chunks.append(idxf.astype(jnp.int32))
    idx_ref[...] = jnp.reshape(
        jnp.concatenate(chunks, axis=0), idx_ref.shape
    )


def _nearest_code_indices(xn, flat, embeddings, *, interpret=False):
    m, k = flat.shape
    n = embeddings.shape[1]
    rows_per_step = M_TILE // _IDX_CHUNK
    return pl.pallas_call(
        _argmin_body,
        grid=(m // M_TILE,),
        in_specs=[
            pl.BlockSpec((M_TILE, 1), lambda i: (i, 0)),
            pl.BlockSpec((M_TILE, k), lambda i: (i, 0)),
            pl.BlockSpec((k, n), lambda i: (0, 0)),
        ],
        out_specs=[
            pl.BlockSpec((rows_per_step, _IDX_CHUNK), lambda i: (i, 0)),
            pl.BlockSpec((n, k), lambda i: (0, 0)),
        ],
        out_shape=[
            jax.ShapeDtypeStruct((m // _IDX_CHUNK, _IDX_CHUNK), jnp.int32),
            jax.ShapeDtypeStruct((n, k), jnp.float32),
        ],
        scratch_shapes=[pltpu.VMEM((1, n), jnp.float32)],
        interpret=interpret,
    )(xn, flat, embeddings)


def _codebook_lookup(table, idx_rows):
    b_total = idx_rows.shape[0] * idx_rows.shape[1]
    d = table.shape[1]
    b_per_w = b_total // _NW
    chunks = b_per_w // _IDX_CHUNK
    mesh = plsc.VectorSubcoreMesh(core_axis_name="c", subcore_axis_name="s")

    @functools.partial(
        pl.kernel,
        mesh=mesh,
        out_type=jax.ShapeDtypeStruct((b_total, d), jnp.float32),
        scratch_types=[
            pltpu.VMEM((chunks, _IDX_CHUNK), jnp.int32),
            pltpu.VMEM((b_per_w, d), jnp.float32),
            pltpu.SemaphoreType.DMA,
        ],
        compiler_params=pltpu.CompilerParams(use_tc_tiling_on_sc=False),
    )
    def gather_kernel(table_hbm, idx_hbm, out_hbm, idx_v, rows_v, sem):
        wid = lax.axis_index("s") * _SC_CORES + lax.axis_index("c")
        pltpu.sync_copy(idx_hbm.at[pl.ds(wid * chunks, chunks)], idx_v)
        copies = [
            pltpu.async_copy(
                table_hbm.at[idx_v.at[j]],
                rows_v.at[pl.ds(j * _IDX_CHUNK, _IDX_CHUNK)],
                sem,
            )
            for j in range(chunks)
        ]
        for c in copies:
            c.wait()
        pltpu.sync_copy(rows_v, out_hbm.at[pl.ds(wid * b_per_w, b_per_w)])

    return gather_kernel(table, idx_rows)


def kernel(x, embeddings):
    input_shape = x.shape
    flat = jnp.reshape(x, (-1, EMBEDDING_DIM))
    xn = jnp.sum(flat ** 2, axis=1, keepdims=True)
    idx_rows, table = _nearest_code_indices(xn, flat, embeddings)
    quantized = _codebook_lookup(table, idx_rows)
    return jnp.reshape(quantized, input_shape)


# speedup vs baseline: 1.0395x; 1.0395x over previous
import functools

import jax
import jax.numpy as jnp
from jax import lax
from jax.experimental import pallas as pl
from jax.experimental.pallas import tpu as pltpu
from jax.experimental.pallas import tpu_sc as plsc

EMBEDDING_DIM = 32
M_TILE = 1024

_SC_CORES = 2
_SC_SUBCORES = 16
_NW = _SC_CORES * _SC_SUBCORES
_IDX_CHUNK = 128


_ROW_CHUNK = 64
_COL_GROUP = 128


def _argmin_body(xn_ref, x_ref, e_ref, idx_ref, table_ref, en_scr):
    @pl.when(pl.program_id(0) == 0)
    def _():
        e = e_ref[...]
        en_scr[...] = jnp.sum(e * e, axis=0, keepdims=True)
        table_ref[...] = e.T

    sim2 = jnp.dot(
        x_ref[...] * -2.0, e_ref[...], preferred_element_type=jnp.float32
    )
    n = sim2.shape[1]
    groups = n // _COL_GROUP
    en = en_scr[...]
    lanef = lax.broadcasted_iota(jnp.int32, (1, _COL_GROUP), 1).astype(
        jnp.float32
    )
    chunks = []
    for c in range(M_TILE // _ROW_CHUNK):
        r0 = c * _ROW_CHUNK
        xn_c = xn_ref[pl.ds(r0, _ROW_CHUNK), :]
        runmin = (xn_c + en[:, :_COL_GROUP]) + sim2[r0:r0 + _ROW_CHUNK, :_COL_GROUP]
        runidx = jnp.zeros((_ROW_CHUNK, _COL_GROUP), jnp.float32)
        for j in range(1, groups):
            c0 = j * _COL_GROUP
            d = (xn_c + en[:, c0:c0 + _COL_GROUP]) + sim2[
                r0:r0 + _ROW_CHUNK, c0:c0 + _COL_GROUP
            ]
            better = d < runmin
            runmin = jnp.where(better, d, runmin)
            runidx = jnp.where(better, jnp.float32(j), runidx)
        m = jnp.min(runmin, axis=1, keepdims=True)
        cand = jnp.where(
            runmin == m, runidx * float(_COL_GROUP) + lanef, jnp.float32(3.0e38)
        )
        idxf = jnp.min(cand, axis=1, keepdims=True)
        chunks.append(idxf.astype(jnp.int32))
    idx_ref[...] = jnp.concatenate(chunks, axis=0)


def _nearest_code_indices(xn, flat, embeddings, *, interpret=False):
    m, k = flat.shape
    n = embeddings.shape[1]
    rows_per_step = M_TILE // _IDX_CHUNK
    return pl.pallas_call(
        _argmin_body,
        grid=(m // M_TILE,),
        in_specs=[
            pl.BlockSpec((M_TILE, 1), lambda i: (i, 0)),
            pl.BlockSpec((M_TILE, k), lambda i: (i, 0)),
            pl.BlockSpec((k, n), lambda i: (0, 0)),
        ],
        out_specs=[
            pl.BlockSpec((M_TILE, 1), lambda i: (i, 0)),
            pl.BlockSpec((n, k), lambda i: (0, 0)),
        ],
        out_shape=[
            jax.ShapeDtypeStruct((m, 1), jnp.int32),
            jax.ShapeDtypeStruct((n, k), jnp.float32),
        ],
        scratch_shapes=[pltpu.VMEM((1, n), jnp.float32)],
        interpret=interpret,
    )(xn, flat, embeddings)


def _codebook_lookup(table, idx_rows):
    b_total = idx_rows.shape[0] * idx_rows.shape[1]
    d = table.shape[1]
    b_per_w = b_total // _NW
    chunks = b_per_w // _IDX_CHUNK
    mesh = plsc.VectorSubcoreMesh(core_axis_name="c", subcore_axis_name="s")

    @functools.partial(
        pl.kernel,
        mesh=mesh,
        out_type=jax.ShapeDtypeStruct((b_total, d), jnp.float32),
        scratch_types=[
            pltpu.VMEM((chunks, _IDX_CHUNK), jnp.int32),
            pltpu.VMEM((b_per_w, d), jnp.float32),
            pltpu.SemaphoreType.DMA,
        ],
        compiler_params=pltpu.CompilerParams(use_tc_tiling_on_sc=False),
    )
    def gather_kernel(table_hbm, idx_hbm, out_hbm, idx_v, rows_v, sem):
        wid = lax.axis_index("s") * _SC_CORES + lax.axis_index("c")
        pltpu.sync_copy(idx_hbm.at[pl.ds(wid * chunks, chunks)], idx_v)
        copies = [
            pltpu.async_copy(
                table_hbm.at[idx_v.at[j]],
                rows_v.at[pl.ds(j * _IDX_CHUNK, _IDX_CHUNK)],
                sem,
            )
            for j in range(chunks)
        ]
        for c in copies:
            c.wait()
        pltpu.sync_copy(rows_v, out_hbm.at[pl.ds(wid * b_per_w, b_per_w)])

    return gather_kernel(table, idx_rows)


def kernel(x, embeddings):
    input_shape = x.shape
    flat = jnp.reshape(x, (-1, EMBEDDING_DIM))
    xn = jnp.sum(flat ** 2, axis=1, keepdims=True)
    idx, table = _nearest_code_indices(xn, flat, embeddings)
    idx_rows = jnp.reshape(idx, (-1, _IDX_CHUNK))
    quantized = _codebook_lookup(table, idx_rows)
    return jnp.reshape(quantized, input_shape)


# speedup vs baseline: 1.0979x; 1.0562x over previous
import functools

import jax
import jax.numpy as jnp
from jax import lax
from jax.experimental import pallas as pl
from jax.experimental.pallas import tpu as pltpu
from jax.experimental.pallas import tpu_sc as plsc

EMBEDDING_DIM = 32
M_TILE = 1024

_SC_CORES = 2
_SC_SUBCORES = 16
_NW = _SC_CORES * _SC_SUBCORES
_IDX_CHUNK = 128


_ROW_CHUNK = 64
_COL_GROUP = 128


def _argmin_body(x_ref, e_ref, idx_ref, en_scr):
    @pl.when(pl.program_id(0) == 0)
    def _():
        e = e_ref[...]
        en_scr[...] = jnp.sum(e * e, axis=0, keepdims=True)

    x = x_ref[...]
    xn = jnp.sum(x * x, axis=1, keepdims=True)
    sim2 = jnp.dot(x * -2.0, e_ref[...], preferred_element_type=jnp.float32)
    n = sim2.shape[1]
    groups = n // _COL_GROUP
    en = en_scr[...]
    lanef = lax.broadcasted_iota(jnp.int32, (1, _COL_GROUP), 1).astype(
        jnp.float32
    )
    chunks = []
    for c in range(M_TILE // _ROW_CHUNK):
        r0 = c * _ROW_CHUNK
        xn_c = xn[r0:r0 + _ROW_CHUNK, :]
        runmin = (xn_c + en[:, :_COL_GROUP]) + sim2[r0:r0 + _ROW_CHUNK, :_COL_GROUP]
        runidx = jnp.zeros((_ROW_CHUNK, _COL_GROUP), jnp.float32)
        for j in range(1, groups):
            c0 = j * _COL_GROUP
            d = (xn_c + en[:, c0:c0 + _COL_GROUP]) + sim2[
                r0:r0 + _ROW_CHUNK, c0:c0 + _COL_GROUP
            ]
            better = d < runmin
            runmin = jnp.where(better, d, runmin)
            runidx = jnp.where(better, jnp.float32(j), runidx)
        m = jnp.min(runmin, axis=1, keepdims=True)
        cand = jnp.where(
            runmin == m, runidx * float(_COL_GROUP) + lanef, jnp.float32(3.0e38)
        )
        idxf = jnp.min(cand, axis=1, keepdims=True)
        chunks.append(idxf.astype(jnp.int32))
    idx_ref[...] = jnp.concatenate(chunks, axis=0)


def _nearest_code_indices(flat, embeddings, *, interpret=False):
    m, k = flat.shape
    n = embeddings.shape[1]
    return pl.pallas_call(
        _argmin_body,
        grid=(m // M_TILE,),
        in_specs=[
            pl.BlockSpec((M_TILE, k), lambda i: (i, 0)),
            pl.BlockSpec((k, n), lambda i: (0, 0)),
        ],
        out_specs=pl.BlockSpec((M_TILE, 1), lambda i: (i, 0)),
        out_shape=jax.ShapeDtypeStruct((m, 1), jnp.int32),
        scratch_shapes=[pltpu.VMEM((1, n), jnp.float32)],
        interpret=interpret,
    )(flat, embeddings)


def _codebook_lookup(table, idx_rows):
    b_total = idx_rows.shape[0] * idx_rows.shape[1]
    d = table.shape[1]
    b_per_w = b_total // _NW
    chunks = b_per_w // _IDX_CHUNK
    mesh = plsc.VectorSubcoreMesh(core_axis_name="c", subcore_axis_name="s")

    @functools.partial(
        pl.kernel,
        mesh=mesh,
        out_type=jax.ShapeDtypeStruct((b_total, d), jnp.float32),
        scratch_types=[
            pltpu.VMEM((chunks, _IDX_CHUNK), jnp.int32),
            pltpu.VMEM((b_per_w, d), jnp.float32),
            pltpu.SemaphoreType.DMA,
        ],
        compiler_params=pltpu.CompilerParams(use_tc_tiling_on_sc=False),
    )
    def gather_kernel(table_hbm, idx_hbm, out_hbm, idx_v, rows_v, sem):
        wid = lax.axis_index("s") * _SC_CORES + lax.axis_index("c")
        pltpu.sync_copy(idx_hbm.at[pl.ds(wid * chunks, chunks)], idx_v)
        copies = [
            pltpu.async_copy(
                table_hbm.at[idx_v.at[j]],
                rows_v.at[pl.ds(j * _IDX_CHUNK, _IDX_CHUNK)],
                sem,
            )
            for j in range(chunks)
        ]
        for c in copies:
            c.wait()
        pltpu.sync_copy(rows_v, out_hbm.at[pl.ds(wid * b_per_w, b_per_w)])

    return gather_kernel(table, idx_rows)


def kernel(x, embeddings):
    input_shape = x.shape
    flat = jnp.reshape(x, (-1, EMBEDDING_DIM))
    idx = _nearest_code_indices(flat, embeddings)
    idx_rows = jnp.reshape(idx, (-1, _IDX_CHUNK))
    quantized = _codebook_lookup(embeddings.T, idx_rows)
    return jnp.reshape(quantized, input_shape)


# speedup vs baseline: 1.1323x; 1.0313x over previous
import functools

import jax
import jax.numpy as jnp
from jax import lax
from jax.experimental import pallas as pl
from jax.experimental.pallas import tpu as pltpu
from jax.experimental.pallas import tpu_sc as plsc

EMBEDDING_DIM = 32
M_TILE = 1024

_SC_CORES = 2
_SC_SUBCORES = 16
_NW = _SC_CORES * _SC_SUBCORES
_IDX_CHUNK = 128


_ROW_CHUNK = 64
_COL_GROUP = 128


def _argmin_body(x_ref, e_ref, idx_ref, en_scr):
    @pl.when(pl.program_id(0) == 0)
    def _():
        e = e_ref[...]
        en_scr[...] = jnp.sum(e * e, axis=0, keepdims=True)

    x = x_ref[...]
    xn = jnp.sum(x * x, axis=1, keepdims=True)
    sim2 = jnp.dot(x * -2.0, e_ref[...], preferred_element_type=jnp.float32)
    n = sim2.shape[1]
    groups = n // _COL_GROUP
    en = en_scr[...]
    lanef = lax.broadcasted_iota(jnp.int32, (1, _COL_GROUP), 1).astype(
        jnp.float32
    )
    chunks = []
    for c in range(M_TILE // _ROW_CHUNK):
        r0 = c * _ROW_CHUNK
        xn_c = xn[r0:r0 + _ROW_CHUNK, :]
        runmin = (xn_c + en[:, :_COL_GROUP]) + sim2[r0:r0 + _ROW_CHUNK, :_COL_GROUP]
        runidx = jnp.zeros((_ROW_CHUNK, _COL_GROUP), jnp.float32)
        for j in range(1, groups):
            c0 = j * _COL_GROUP
            d = (xn_c + en[:, c0:c0 + _COL_GROUP]) + sim2[
                r0:r0 + _ROW_CHUNK, c0:c0 + _COL_GROUP
            ]
            better = d < runmin
            runmin = jnp.minimum(d, runmin)
            runidx = jnp.where(better, jnp.float32(j), runidx)
        m = jnp.min(runmin, axis=1, keepdims=True)
        cand = jnp.where(
            runmin == m, runidx * float(_COL_GROUP) + lanef, jnp.float32(3.0e38)
        )
        idxf = jnp.min(cand, axis=1, keepdims=True)
        chunks.append(idxf.astype(jnp.int32))
    idx_ref[...] = jnp.concatenate(chunks, axis=0)


def _nearest_code_indices(flat, embeddings, *, interpret=False):
    m, k = flat.shape
    n = embeddings.shape[1]
    return pl.pallas_call(
        _argmin_body,
        grid=(m // M_TILE,),
        in_specs=[
            pl.BlockSpec((M_TILE, k), lambda i: (i, 0)),
            pl.BlockSpec((k, n), lambda i: (0, 0)),
        ],
        out_specs=pl.BlockSpec((M_TILE, 1), lambda i: (i, 0)),
        out_shape=jax.ShapeDtypeStruct((m, 1), jnp.int32),
        scratch_shapes=[pltpu.VMEM((1, n), jnp.float32)],
        interpret=interpret,
    )(flat, embeddings)


def _codebook_lookup(table, idx_rows):
    b_total = idx_rows.shape[0] * idx_rows.shape[1]
    d = table.shape[1]
    b_per_w = b_total // _NW
    chunks = b_per_w // _IDX_CHUNK
    mesh = plsc.VectorSubcoreMesh(core_axis_name="c", subcore_axis_name="s")

    @functools.partial(
        pl.kernel,
        mesh=mesh,
        out_type=jax.ShapeDtypeStruct((b_total, d), jnp.float32),
        scratch_types=[
            pltpu.VMEM((chunks, _IDX_CHUNK), jnp.int32),
            pltpu.VMEM((b_per_w, d), jnp.float32),
            pltpu.SemaphoreType.DMA,
        ],
        compiler_params=pltpu.CompilerParams(use_tc_tiling_on_sc=False),
    )
    def gather_kernel(table_hbm, idx_hbm, out_hbm, idx_v, rows_v, sem):
        wid = lax.axis_index("s") * _SC_CORES + lax.axis_index("c")
        pltpu.sync_copy(idx_hbm.at[pl.ds(wid * chunks, chunks)], idx_v)
        copies = [
            pltpu.async_copy(
                table_hbm.at[idx_v.at[j]],
                rows_v.at[pl.ds(j * _IDX_CHUNK, _IDX_CHUNK)],
                sem,
            )
            for j in range(chunks)
        ]
        for c in copies:
            c.wait()
        pltpu.sync_copy(rows_v, out_hbm.at[pl.ds(wid * b_per_w, b_per_w)])

    return gather_kernel(table, idx_rows)


def kernel(x, embeddings):
    input_shape = x.shape
    flat = jnp.reshape(x, (-1, EMBEDDING_DIM))
    idx = _nearest_code_indices(flat, embeddings)
    idx_rows = jnp.reshape(idx, (-1, _IDX_CHUNK))
    quantized = _codebook_lookup(embeddings.T, idx_rows)
    return jnp.reshape(quantized, input_shape)
